# baseline (speedup 1.0000x reference)
import functools

import jax
import jax.numpy as jnp
from jax import lax
from jax.experimental import pallas as pl
from jax.experimental.pallas import tpu as pltpu
from jax.experimental.pallas import tpu_sc as plsc

B, C, H, W = 16, 2, 512, 512
NPIX = B * H * W
NC, NS = 2, 16
NW = NC * NS

B_SC = 6
RPW = B_SC * 16
RBLK = 16
NCHUNK = RPW // RBLK
NPAIR = NCHUNK // 2
GRPW = W // 16

B_TC = B - B_SC
TCROWS = 128
TCQ = H // TCROWS

LN2 = 0.6931471805599453
_C = (0.15551455, -1.0395688, 3.0299323, -129.14516)


def _group_loss(l0, l1, t):
    z = l1 - l0
    zs = lax.bitcast_convert_type(
        lax.bitcast_convert_type(z, jnp.int32) ^ (t << 31), jnp.float32)
    e = jnp.exp(zs)
    p = 1.0 / (1.0 + e)
    p = jnp.maximum(p, 1e-8)
    omega = 1.0 - p
    f = omega * omega
    bits = lax.bitcast_convert_type(p, jnp.int32)
    exf = (bits >> 23).astype(jnp.float32)
    m = lax.bitcast_convert_type((bits & 0x007FFFFF) | 0x3F800000, jnp.float32)
    poly = _C[0]
    for c in _C[1:]:
        poly = poly * m + c
    lp = exf + poly
    return f * lp


def _focal_body(preds_hbm, trues_hbm, out_hbm,
                l0a, l1a, ta, l0b, l1b, tb, accv,
                s0a, s1a, sta, s0b, s1b, stb):
    wid = lax.axis_index("s") * NC + lax.axis_index("c")
    g0 = wid * RPW

    def start(k, bufs, sems):
        l0v, l1v, tv = bufs
        sl0, sl1, st = sems
        g = g0 + k * RBLK
        b = g // H
        r = g - b * H
        pltpu.async_copy(preds_hbm.at[b, 0, pl.ds(r, RBLK), :], l0v, sl0)
        pltpu.async_copy(preds_hbm.at[b, 1, pl.ds(r, RBLK), :], l1v, sl1)
        pltpu.async_copy(trues_hbm.at[b, pl.ds(r, RBLK), :], tv, st)

    def wait(bufs, sems):
        l0v, l1v, tv = bufs
        sl0, sl1, st = sems
        pltpu.make_async_copy(preds_hbm.at[0, 0, pl.ds(0, RBLK), :],
                              l0v, sl0).wait()
        pltpu.make_async_copy(preds_hbm.at[0, 1, pl.ds(0, RBLK), :],
                              l1v, sl1).wait()
        pltpu.make_async_copy(trues_hbm.at[0, pl.ds(0, RBLK), :],
                              tv, st).wait()

    def compute(bufs, acc):
        l0v, l1v, tv = bufs

        def row_body(i, a):
            def grp_body(j, aa):
                s = j * 16
                return aa + _group_loss(l0v[i, pl.ds(s, 16)],
                                        l1v[i, pl.ds(s, 16)],
                                        tv[i, pl.ds(s, 16)])

            return lax.fori_loop(0, GRPW, grp_body, a, unroll=8)

        return lax.fori_loop(0, RBLK, row_body, acc)

    slot0 = (l0a, l1a, ta)
    slot1 = (l0b, l1b, tb)
    sem0 = (s0a, s1a, sta)
    sem1 = (s0b, s1b, stb)

    start(0, slot0, sem0)

    def pair_body(kk, acc):
        start(2 * kk + 1, slot1, sem1)
        wait(slot0, sem0)
        acc = compute(slot0, acc)

        @pl.when(kk < NPAIR - 1)
        def _():
            start(2 * kk + 2, slot0, sem0)

        wait(slot1, sem1)
        return compute(slot1, acc)

    acc = lax.fori_loop(0, NPAIR, pair_body, jnp.zeros((16,), jnp.float32))
    accv[...] = acc
    pltpu.sync_copy(accv, out_hbm.at[wid])


_focal_sc = functools.partial(
    pl.kernel,
    out_type=jax.ShapeDtypeStruct((NW, 16), jnp.float32),
    mesh=plsc.VectorSubcoreMesh(core_axis_name="c", subcore_axis_name="s",
                                num_cores=NC, num_subcores=NS),
    compiler_params=pltpu.CompilerParams(use_tc_tiling_on_sc=True),
    scratch_types=[
        pltpu.VMEM((RBLK, W), jnp.float32),
        pltpu.VMEM((RBLK, W), jnp.float32),
        pltpu.VMEM((RBLK, W), jnp.int32),
        pltpu.VMEM((RBLK, W), jnp.float32),
        pltpu.VMEM((RBLK, W), jnp.float32),
        pltpu.VMEM((RBLK, W), jnp.int32),
        pltpu.VMEM((16,), jnp.float32),
        pltpu.SemaphoreType.DMA,
        pltpu.SemaphoreType.DMA,
        pltpu.SemaphoreType.DMA,
        pltpu.SemaphoreType.DMA,
        pltpu.SemaphoreType.DMA,
        pltpu.SemaphoreType.DMA,
    ],
)(_focal_body)


def _tc_body(pr, tr, o):
    first = (pl.program_id(0) == 0) & (pl.program_id(1) == 0)

    @pl.when(first)
    def _():
        o[...] = jnp.zeros((8, 128), jnp.float32)

    l0 = pr[0, 0]
    l1 = pr[0, 1]
    t = tr[0]
    z = l1 - l0
    zs = jnp.where(t > 0, -z, z)
    p = 1.0 / (1.0 + jnp.exp(zs))
    p = jnp.clip(p, 1e-8, 1.0 - 1e-8)
    omega = 1.0 - p
    s = jnp.sum(omega * omega * jnp.log(p))
    o[...] += jnp.full((8, 128), s, jnp.float32)


_focal_tc = pl.pallas_call(
    _tc_body,
    grid=(B_TC, TCQ),
    in_specs=[
        pl.BlockSpec((1, C, TCROWS, W), lambda i, q: (B_SC + i, 0, q, 0)),
        pl.BlockSpec((1, TCROWS, W), lambda i, q: (B_SC + i, q, 0)),
    ],
    out_specs=pl.BlockSpec((8, 128), lambda i, q: (0, 0)),
    out_shape=jax.ShapeDtypeStruct((8, 128), jnp.float32),
)


@jax.jit
def kernel(preds, trues):
    ti = trues.astype(jnp.int32)
    sc_part = _focal_sc(preds, ti)
    tc_part = _focal_tc(preds, ti)
    total = LN2 * jnp.sum(sc_part, dtype=jnp.float32) + tc_part[0, 0]
    return -total / NPIX


# speedup vs baseline: 1.0624x; 1.0624x over previous
import functools

import jax
import jax.numpy as jnp
from jax import lax
from jax.experimental import pallas as pl
from jax.experimental.pallas import tpu as pltpu
from jax.experimental.pallas import tpu_sc as plsc

B, C, H, W = 16, 2, 512, 512
NPIX = B * H * W
NC, NS = 2, 16
NW = NC * NS

B_SC = 8
SECT = NW // B_SC
RPS = H // SECT
RBLK = 32
NCHUNK = RPS // RBLK
NPAIR = NCHUNK // 2
GRPW = W // 16

B_TC = B - B_SC
TCROWS = 128
TCQ = H // TCROWS

LN2 = 0.6931471805599453
_C = (0.15551455, -1.0395688, 3.0299323, -129.14516)


def _group_loss(l0, l1, t):
    z = l1 - l0
    zs = lax.bitcast_convert_type(
        lax.bitcast_convert_type(z, jnp.int32) ^ (t << 31), jnp.float32)
    e = jnp.exp(zs)
    p = 1.0 / (1.0 + e)
    p = jnp.maximum(p, 1e-8)
    omega = 1.0 - p
    f = omega * omega
    bits = lax.bitcast_convert_type(p, jnp.int32)
    exf = (bits >> 23).astype(jnp.float32)
    m = lax.bitcast_convert_type((bits & 0x007FFFFF) | 0x3F800000, jnp.float32)
    poly = _C[0]
    for c in _C[1:]:
        poly = poly * m + c
    lp = exf + poly
    return f * lp


def _focal_body(preds_hbm, trues_hbm, out_hbm,
                l0a, l1a, ta, l0b, l1b, tb, accv,
                s0a, s1a, sta, s0b, s1b, stb):
    wid = lax.axis_index("s") * NC + lax.axis_index("c")
    b = wid // SECT
    row0 = (wid % SECT) * RPS

    def start(k, bufs, sems):
        l0v, l1v, tv = bufs
        sl0, sl1, st = sems
        r = row0 + k * RBLK
        pltpu.async_copy(preds_hbm.at[b, 0, pl.ds(r, RBLK), :], l0v, sl0)
        pltpu.async_copy(preds_hbm.at[b, 1, pl.ds(r, RBLK), :], l1v, sl1)
        pltpu.async_copy(trues_hbm.at[b, pl.ds(r, RBLK), :], tv, st)

    def wait(bufs, sems):
        l0v, l1v, tv = bufs
        sl0, sl1, st = sems
        pltpu.make_async_copy(preds_hbm.at[0, 0, pl.ds(0, RBLK), :],
                              l0v, sl0).wait()
        pltpu.make_async_copy(preds_hbm.at[0, 1, pl.ds(0, RBLK), :],
                              l1v, sl1).wait()
        pltpu.make_async_copy(trues_hbm.at[0, pl.ds(0, RBLK), :],
                              tv, st).wait()

    def compute(bufs, acc):
        l0v, l1v, tv = bufs

        def row_body(i, a):
            def grp_body(j, aa):
                s = j * 16
                return aa + _group_loss(l0v[i, pl.ds(s, 16)],
                                        l1v[i, pl.ds(s, 16)],
                                        tv[i, pl.ds(s, 16)])

            return lax.fori_loop(0, GRPW, grp_body, a, unroll=8)

        return lax.fori_loop(0, RBLK, row_body, acc)

    slot0 = (l0a, l1a, ta)
    slot1 = (l0b, l1b, tb)
    sem0 = (s0a, s1a, sta)
    sem1 = (s0b, s1b, stb)

    start(0, slot0, sem0)

    def pair_body(kk, acc):
        start(2 * kk + 1, slot1, sem1)
        wait(slot0, sem0)
        acc = compute(slot0, acc)

        @pl.when(kk < NPAIR - 1)
        def _():
            start(2 * kk + 2, slot0, sem0)

        wait(slot1, sem1)
        return compute(slot1, acc)

    acc = lax.fori_loop(0, NPAIR, pair_body, jnp.zeros((16,), jnp.float32))
    accv[...] = acc
    pltpu.sync_copy(accv, out_hbm.at[wid])


_focal_sc = functools.partial(
    pl.kernel,
    out_type=jax.ShapeDtypeStruct((NW, 16), jnp.float32),
    mesh=plsc.VectorSubcoreMesh(core_axis_name="c", subcore_axis_name="s",
                                num_cores=NC, num_subcores=NS),
    compiler_params=pltpu.CompilerParams(use_tc_tiling_on_sc=True),
    scratch_types=[
        pltpu.VMEM((RBLK, W), jnp.float32),
        pltpu.VMEM((RBLK, W), jnp.float32),
        pltpu.VMEM((RBLK, W), jnp.int32),
        pltpu.VMEM((RBLK, W), jnp.float32),
        pltpu.VMEM((RBLK, W), jnp.float32),
        pltpu.VMEM((RBLK, W), jnp.int32),
        pltpu.VMEM((16,), jnp.float32),
        pltpu.SemaphoreType.DMA,
        pltpu.SemaphoreType.DMA,
        pltpu.SemaphoreType.DMA,
        pltpu.SemaphoreType.DMA,
        pltpu.SemaphoreType.DMA,
        pltpu.SemaphoreType.DMA,
    ],
)(_focal_body)


def _tc_body(pr, tr, o):
    first = (pl.program_id(0) == 0) & (pl.program_id(1) == 0)

    @pl.when(first)
    def _():
        o[...] = jnp.zeros((8, 128), jnp.float32)

    l0 = pr[0, 0]
    l1 = pr[0, 1]
    t = tr[0]
    z = l1 - l0
    zs = jnp.where(t > 0, -z, z)
    p = 1.0 / (1.0 + jnp.exp(zs))
    p = jnp.clip(p, 1e-8, 1.0 - 1e-8)
    omega = 1.0 - p
    s = jnp.sum(omega * omega * jnp.log(p))
    o[...] += jnp.full((8, 128), s, jnp.float32)


_focal_tc = pl.pallas_call(
    _tc_body,
    grid=(B_TC, TCQ),
    in_specs=[
        pl.BlockSpec((1, C, TCROWS, W), lambda i, q: (B_SC + i, 0, q, 0)),
        pl.BlockSpec((1, TCROWS, W), lambda i, q: (B_SC + i, q, 0)),
    ],
    out_specs=pl.BlockSpec((8, 128), lambda i, q: (0, 0)),
    out_shape=jax.ShapeDtypeStruct((8, 128), jnp.float32),
)


@jax.jit
def kernel(preds, trues):
    ti = trues.astype(jnp.int32)
    sc_part = _focal_sc(preds, ti)
    tc_part = _focal_tc(preds, ti)
    total = LN2 * jnp.sum(sc_part, dtype=jnp.float32) + tc_part[0, 0]
    return -total / NPIX
